# baseline (speedup 1.0000x reference)
import jax
import jax.numpy as jnp
from jax.experimental import pallas as pl
from jax.experimental.pallas import tpu as pltpu

N_PROXIES = 8192
N_CAMS = 8
PPC = N_PROXIES // N_CAMS
TEMP = 0.07
B = 1024
D = 256
TILE = 256
P = 3072
NT = P // TILE


def _tile_kernel(scal_ref, feat_ref, mem_ref, cam_ref, tgt_ref, out_ref,
                 pos_s, winv_s, tgtf_s, cnt_s, tb_s):
    c = pl.program_id(0)
    feat = feat_ref[...]

    @pl.when(c == 0)
    def _scalar_setup():
        def zero(i, _):
            cnt_s[i] = 0
            return 0
        jax.lax.fori_loop(0, N_CAMS, zero, 0)

        def hist(i, _):
            v = scal_ref[i]
            cnt_s[v] = cnt_s[v] + 1
            return 0
        jax.lax.fori_loop(0, B, hist, 0)
        tb_s[0] = 0

        def prefix(i, _):
            tb_s[i + 1] = tb_s[i] + (cnt_s[i] + TILE - 1) // TILE
            return 0
        jax.lax.fori_loop(0, N_CAMS, prefix, 0)

    @pl.when(c == 0)
    def _setup():
        camv = cam_ref[...]
        camsub = jax.lax.broadcasted_iota(jnp.int32, (N_CAMS, B), 0)
        ohi = (camsub == camv).astype(jnp.float32)
        cnt = jnp.sum(ohi, axis=1, keepdims=True)
        padded = jnp.floor((cnt + (TILE - 1)) * (1.0 / TILE)) * TILE
        r8 = jax.lax.broadcasted_iota(jnp.int32, (N_CAMS, N_CAMS), 0)
        c8 = jax.lax.broadcasted_iota(jnp.int32, (N_CAMS, N_CAMS), 1)
        strict_lt = (c8 < r8).astype(jnp.float32)
        starts = jax.lax.dot_general(
            strict_lt, padded, (((1,), (0,)), ((), ())),
            preferred_element_type=jnp.float32)
        ri = jax.lax.broadcasted_iota(jnp.int32, (B, B), 0)
        ci = jax.lax.broadcasted_iota(jnp.int32, (B, B), 1)
        lt_inc = (ri <= ci).astype(jnp.float32)
        incl = jax.lax.dot_general(
            ohi, lt_inc, (((1,), (0,)), ((), ())),
            preferred_element_type=jnp.float32)
        rank = jnp.sum(ohi * (incl - 1.0), axis=0, keepdims=True)
        pos_s[...] = jnp.sum(ohi * starts, axis=0, keepdims=True) + rank
        winv_s[...] = jnp.sum(jnp.where(ohi > 0, 1.0 / cnt, 0.0), axis=0,
                              keepdims=True)
        tgtf_s[...] = (tgt_ref[...] - camv * PPC).astype(jnp.float32)
        out_ref[...] = jnp.zeros_like(out_ref)

    pos = pos_s[...]
    winv = winv_s[...]
    tgtf = tgtf_s[...]
    w = mem_ref[...]

    def body(t, acc):
        pj = (jax.lax.broadcasted_iota(jnp.int32, (TILE, 1), 0)
              + t * TILE).astype(jnp.float32)
        gb = pos == pj
        g = gb.astype(jnp.float32)
        x = jax.lax.dot_general(
            g, feat, (((1,), (0,)), ((), ())),
            preferred_element_type=jnp.float32)
        tgt_t = jnp.sum(jnp.where(gb, tgtf, 0.0), axis=1, keepdims=True)
        w_t = jnp.sum(jnp.where(gb, winv, 0.0), axis=1, keepdims=True)
        sim = jax.lax.dot_general(
            x, w, (((1,), (1,)), ((), ())), preferred_element_type=jnp.float32
        ) * (1.0 / TEMP)
        lse = jnp.log(jnp.sum(jnp.exp(sim), axis=1, keepdims=True))
        cols = jax.lax.broadcasted_iota(jnp.int32, (TILE, PPC), 1)
        tlogit = jnp.sum(jnp.where(cols == tgt_t.astype(jnp.int32), sim, 0.0),
                         axis=1, keepdims=True)
        return acc + (lse - tlogit) * w_t

    acc = jax.lax.fori_loop(tb_s[c], tb_s[c + 1], body,
                            jnp.zeros((TILE, 1), jnp.float32))
    lane = jax.lax.broadcasted_iota(jnp.int32, (1, 128), 1)
    out_ref[...] += jnp.where(lane == 0, jnp.sum(acc), 0.0)


def kernel(batch_feat, abs_proxy_label, camid, pseudo_cluster_label, memory,
           epoch, k, inter_loss_epoch):
    camid = camid.astype(jnp.int32)

    out = pl.pallas_call(
        _tile_kernel,
        grid_spec=pltpu.PrefetchScalarGridSpec(
            num_scalar_prefetch=1,
            grid=(N_CAMS,),
            in_specs=[
                pl.BlockSpec((B, D), lambda c, tc: (0, 0)),
                pl.BlockSpec((PPC, D), lambda c, tc: (c, 0)),
                pl.BlockSpec((1, B), lambda c, tc: (0, 0)),
                pl.BlockSpec((1, B), lambda c, tc: (0, 0)),
            ],
            out_specs=pl.BlockSpec((1, 128), lambda c, tc: (0, 0)),
            scratch_shapes=[
                pltpu.VMEM((1, B), jnp.float32),
                pltpu.VMEM((1, B), jnp.float32),
                pltpu.VMEM((1, B), jnp.float32),
                pltpu.SMEM((N_CAMS,), jnp.int32),
                pltpu.SMEM((N_CAMS + 1,), jnp.int32),
            ],
        ),
        out_shape=jax.ShapeDtypeStruct((1, 128), jnp.float32),
    )(camid, batch_feat, memory,
      camid.reshape(1, B), abs_proxy_label.astype(jnp.int32).reshape(1, B))
    return out[0, 0]


# speedup vs baseline: 1.1962x; 1.1962x over previous
import jax
import jax.numpy as jnp
from jax.experimental import pallas as pl
from jax.experimental.pallas import tpu as pltpu

N_PROXIES = 8192
N_CAMS = 8
PPC = N_PROXIES // N_CAMS
TEMP = 0.07
B = 1024
D = 256
TILE = 256
P = 3072
NT = P // TILE


def _tile_kernel(scal_ref, feat_ref, mem_ref, cam_ref, tgt_ref, out_ref,
                 pos_s, winv_s, tgtf_s):
    c = pl.program_id(0)
    feat = feat_ref[...]

    @pl.when(c == 0)
    def _setup():
        camv = cam_ref[...]
        camsub = jax.lax.broadcasted_iota(jnp.int32, (N_CAMS, B), 0)
        ohi = (camsub == camv).astype(jnp.float32)
        cnt = jnp.sum(ohi, axis=1, keepdims=True)
        padded = jnp.floor((cnt + (TILE - 1)) * (1.0 / TILE)) * TILE
        r8 = jax.lax.broadcasted_iota(jnp.int32, (N_CAMS, N_CAMS), 0)
        c8 = jax.lax.broadcasted_iota(jnp.int32, (N_CAMS, N_CAMS), 1)
        strict_lt = (c8 < r8).astype(jnp.float32)
        starts = jax.lax.dot_general(
            strict_lt, padded, (((1,), (0,)), ((), ())),
            preferred_element_type=jnp.float32)
        ri = jax.lax.broadcasted_iota(jnp.int32, (B, B), 0)
        ci = jax.lax.broadcasted_iota(jnp.int32, (B, B), 1)
        lt_inc = (ri <= ci).astype(jnp.float32)
        incl = jax.lax.dot_general(
            ohi, lt_inc, (((1,), (0,)), ((), ())),
            preferred_element_type=jnp.float32)
        rank = jnp.sum(ohi * (incl - 1.0), axis=0, keepdims=True)
        pos_s[...] = jnp.sum(ohi * starts, axis=0, keepdims=True) + rank
        winv_s[...] = jnp.sum(jnp.where(ohi > 0, 1.0 / cnt, 0.0), axis=0,
                              keepdims=True)
        tgtf_s[...] = (tgt_ref[...] - camv * PPC).astype(jnp.float32)
        out_ref[...] = jnp.zeros_like(out_ref)

    pos = pos_s[...]
    winv = winv_s[...]
    tgtf = tgtf_s[...]
    w = mem_ref[...]

    def body(t, acc):
        pj = (jax.lax.broadcasted_iota(jnp.int32, (TILE, 1), 0)
              + t * TILE).astype(jnp.float32)
        gb = pos == pj
        g = gb.astype(jnp.float32)
        x = jax.lax.dot_general(
            g, feat, (((1,), (0,)), ((), ())),
            preferred_element_type=jnp.float32)
        tgt_t = jnp.sum(jnp.where(gb, tgtf, 0.0), axis=1, keepdims=True)
        w_t = jnp.sum(jnp.where(gb, winv, 0.0), axis=1, keepdims=True)
        sim = jax.lax.dot_general(
            x, w, (((1,), (1,)), ((), ())), preferred_element_type=jnp.float32
        ) * (1.0 / TEMP)
        lse = jnp.log(jnp.sum(jnp.exp(sim), axis=1, keepdims=True))
        cols = jax.lax.broadcasted_iota(jnp.int32, (TILE, PPC), 1)
        tlogit = jnp.sum(jnp.where(cols == tgt_t.astype(jnp.int32), sim, 0.0),
                         axis=1, keepdims=True)
        return acc + (lse - tlogit) * w_t

    acc = jax.lax.fori_loop(scal_ref[c], scal_ref[c + 1], body,
                            jnp.zeros((TILE, 1), jnp.float32))
    lane = jax.lax.broadcasted_iota(jnp.int32, (1, 128), 1)
    out_ref[...] += jnp.where(lane == 0, jnp.sum(acc), 0.0)


def kernel(batch_feat, abs_proxy_label, camid, pseudo_cluster_label, memory,
           epoch, k, inter_loss_epoch):
    camid = camid.astype(jnp.int32)

    cams = jnp.arange(N_CAMS, dtype=jnp.int32)
    cnt = jnp.sum((camid[None, :] == cams[:, None]).astype(jnp.int32), axis=1)
    ntiles = (cnt + TILE - 1) // TILE
    tb = jnp.sum(jnp.where(cams[None, :] < cams[:, None], ntiles[None, :], 0),
                 axis=1)
    scalars = jnp.concatenate([tb, (tb[N_CAMS - 1] + ntiles[N_CAMS - 1])[None]]
                              ).astype(jnp.int32)

    out = pl.pallas_call(
        _tile_kernel,
        grid_spec=pltpu.PrefetchScalarGridSpec(
            num_scalar_prefetch=1,
            grid=(N_CAMS,),
            in_specs=[
                pl.BlockSpec((B, D), lambda c, tc: (0, 0)),
                pl.BlockSpec((PPC, D), lambda c, tc: (c, 0)),
                pl.BlockSpec((1, B), lambda c, tc: (0, 0)),
                pl.BlockSpec((1, B), lambda c, tc: (0, 0)),
            ],
            out_specs=pl.BlockSpec((1, 128), lambda c, tc: (0, 0)),
            scratch_shapes=[
                pltpu.VMEM((1, B), jnp.float32),
                pltpu.VMEM((1, B), jnp.float32),
                pltpu.VMEM((1, B), jnp.float32),
            ],
        ),
        out_shape=jax.ShapeDtypeStruct((1, 128), jnp.float32),
    )(scalars, batch_feat, memory,
      camid.reshape(1, B), abs_proxy_label.astype(jnp.int32).reshape(1, B))
    return out[0, 0]


# speedup vs baseline: 1.4124x; 1.1808x over previous
import jax
import jax.numpy as jnp
from jax.experimental import pallas as pl
from jax.experimental.pallas import tpu as pltpu

N_PROXIES = 8192
N_CAMS = 8
PPC = N_PROXIES // N_CAMS
TEMP = 0.07
B = 1024
D = 256
TILE = 256
P = 3072
NT = P // TILE


def _tile_kernel(scal_ref, feat_ref, mem_ref, cam_ref, tgt_ref, out_ref,
                 pos_s, winv_s, tgtf_s):
    c = pl.program_id(0)
    feat = feat_ref[...]

    @pl.when(c == 0)
    def _setup():
        camv = cam_ref[...]
        camsub = jax.lax.broadcasted_iota(jnp.int32, (N_CAMS, B), 0)
        ohi = (camsub == camv).astype(jnp.float32)
        cnt = jnp.sum(ohi, axis=1, keepdims=True)
        padded = jnp.floor((cnt + (TILE - 1)) * (1.0 / TILE)) * TILE
        r8 = jax.lax.broadcasted_iota(jnp.int32, (N_CAMS, N_CAMS), 0)
        c8 = jax.lax.broadcasted_iota(jnp.int32, (N_CAMS, N_CAMS), 1)
        strict_lt = (c8 < r8).astype(jnp.float32)
        starts = jax.lax.dot_general(
            strict_lt, padded, (((1,), (0,)), ((), ())),
            preferred_element_type=jnp.float32)
        ri = jax.lax.broadcasted_iota(jnp.int32, (B, B), 0)
        ci = jax.lax.broadcasted_iota(jnp.int32, (B, B), 1)
        lt_inc = (ri <= ci).astype(jnp.float32)
        incl = jax.lax.dot_general(
            ohi, lt_inc, (((1,), (0,)), ((), ())),
            preferred_element_type=jnp.float32)
        rank = jnp.sum(ohi * (incl - 1.0), axis=0, keepdims=True)
        pos_s[...] = jnp.sum(ohi * starts, axis=0, keepdims=True) + rank
        winv_s[...] = jnp.sum(jnp.where(ohi > 0, 1.0 / cnt, 0.0), axis=0,
                              keepdims=True)
        tgtf_s[...] = (tgt_ref[...] - camv * PPC).astype(jnp.float32)
        out_ref[...] = jnp.zeros_like(out_ref)

    pos = pos_s[...]
    winv = winv_s[...]
    tgtf = tgtf_s[...]
    w = mem_ref[...]

    def body(t, acc):
        pj = (jax.lax.broadcasted_iota(jnp.int32, (TILE, 1), 0)
              + t * TILE).astype(jnp.float32)
        gb = pos == pj
        g = gb.astype(jnp.float32)
        x = jax.lax.dot_general(
            g, feat, (((1,), (0,)), ((), ())),
            preferred_element_type=jnp.float32)
        tgt_t = jnp.sum(jnp.where(gb, tgtf, 0.0), axis=1, keepdims=True)
        w_t = jnp.sum(jnp.where(gb, winv, 0.0), axis=1, keepdims=True)
        sim = jax.lax.dot_general(
            x, w, (((1,), (1,)), ((), ())), preferred_element_type=jnp.float32
        ) * (1.0 / TEMP)
        lse = jnp.log(jnp.sum(jnp.exp(sim), axis=1, keepdims=True))
        cols = jax.lax.broadcasted_iota(jnp.int32, (TILE, PPC), 1)
        tlogit = jnp.sum(jnp.where(cols == tgt_t.astype(jnp.int32), sim, 0.0),
                         axis=1, keepdims=True)
        return acc + (lse - tlogit) * w_t

    acc = jax.lax.fori_loop(scal_ref[c], scal_ref[c + 1], body,
                            jnp.zeros((TILE, 1), jnp.float32))
    lane = jax.lax.broadcasted_iota(jnp.int32, (1, 128), 1)
    out_ref[...] += jnp.where(lane == 0, jnp.sum(acc), 0.0)


def kernel(batch_feat, abs_proxy_label, camid, pseudo_cluster_label, memory,
           epoch, k, inter_loss_epoch):
    camid = camid.astype(jnp.int32)

    cams = jnp.arange(N_CAMS, dtype=jnp.int32)
    cnt = jnp.sum((camid[None, :] == cams[:, None]).astype(jnp.int32), axis=1)
    ntiles = (cnt + TILE - 1) // TILE
    scalars = jnp.sum(
        jnp.where(cams[None, :] < jnp.arange(N_CAMS + 1, dtype=jnp.int32)[:, None],
                  ntiles[None, :], 0), axis=1).astype(jnp.int32)

    out = pl.pallas_call(
        _tile_kernel,
        grid_spec=pltpu.PrefetchScalarGridSpec(
            num_scalar_prefetch=1,
            grid=(N_CAMS,),
            in_specs=[
                pl.BlockSpec((B, D), lambda c, tc: (0, 0)),
                pl.BlockSpec((PPC, D), lambda c, tc: (c, 0)),
                pl.BlockSpec((1, B), lambda c, tc: (0, 0)),
                pl.BlockSpec((1, B), lambda c, tc: (0, 0)),
            ],
            out_specs=pl.BlockSpec((1, 128), lambda c, tc: (0, 0)),
            scratch_shapes=[
                pltpu.VMEM((1, B), jnp.float32),
                pltpu.VMEM((1, B), jnp.float32),
                pltpu.VMEM((1, B), jnp.float32),
            ],
        ),
        out_shape=jax.ShapeDtypeStruct((1, 128), jnp.float32),
    )(scalars, batch_feat, memory,
      camid.reshape(1, B), abs_proxy_label.astype(jnp.int32).reshape(1, B))
    return out[0, 0]
